# Initial kernel scaffold; baseline (speedup 1.0000x reference)
import jax
import jax.numpy as jnp
from jax.experimental import pallas as pl


def kernel(x, edge_index, batch, W_in, b_in, Wg, Wih, Whh, bih, bhh, W1, b1, W2, b2, W3, b3):
    raise NotImplementedError("write your pallas kernel here")


# speedup vs baseline: 5.8981x; 5.8981x over previous
import functools

import jax
import jax.numpy as jnp
from jax import lax
from jax.experimental import pallas as pl
from jax.experimental.pallas import tpu as pltpu
from jax.experimental.pallas import tpu_sc as plsc

_N = 10000
_E = 320000
_D = 128
_H = 64
_G = 256
_NB, _NL = 4, 2

_NC = 2
_NS = 16
_NW = _NC * _NS
_C = 128
_CH = -(-_E // (_NW * _C))
_E_PAD = _NW * _CH * _C
_ACC_N = 10240
_STRIPE = _ACC_N // _NS


def _seg_body(src_hbm, dst_hbm, m_hbm, zero_hbm, out_hbm, srcv, dstv, rows, acc):
    cid = lax.axis_index("c")
    sid = lax.axis_index("s")
    wid = cid * _NS + sid

    pltpu.sync_copy(src_hbm.at[wid], srcv)
    pltpu.sync_copy(dst_hbm.at[wid], dstv)
    pltpu.sync_copy(zero_hbm, acc.at[pl.ds(sid * _STRIPE, _STRIPE)])
    plsc.subcore_barrier()

    def body(c, carry):
        pltpu.sync_copy(m_hbm.at[srcv.at[c]], rows)
        pltpu.sync_copy(rows, acc.at[dstv.at[c]], add=True)
        return carry

    lax.fori_loop(0, _CH, body, 0)
    plsc.subcore_barrier()
    pltpu.sync_copy(acc.at[pl.ds(sid * _STRIPE, _STRIPE)],
                    out_hbm.at[cid, pl.ds(sid * _STRIPE, _STRIPE)])


_seg_sum = pl.kernel(
    _seg_body,
    out_type=jax.ShapeDtypeStruct((_NC, _ACC_N, _H), jnp.float32),
    mesh=plsc.VectorSubcoreMesh(core_axis_name="c", subcore_axis_name="s"),
    scratch_types=[
        pltpu.VMEM((_CH, _C), jnp.int32),
        pltpu.VMEM((_CH, _C), jnp.int32),
        pltpu.VMEM((_C, _H), jnp.float32),
        pltpu.VMEM_SHARED((_ACC_N, _H), jnp.float32),
    ],
    compiler_params=pltpu.CompilerParams(use_tc_tiling_on_sc=False),
)


def _mmT(a, w):
    return lax.dot_general(a, w, (((1,), (1,)), ((), ())),
                           preferred_element_type=jnp.float32)


def _in_proj_body(x_ref, win_ref, bin_ref, wg_ref, h_ref, m_ref):
    h = _mmT(x_ref[...], win_ref[...]) + bin_ref[...]
    h_ref[...] = h
    m_ref[...] = jnp.dot(h, wg_ref[...], preferred_element_type=jnp.float32)


_in_proj = pl.pallas_call(
    _in_proj_body,
    out_shape=(jax.ShapeDtypeStruct((_N, _H), jnp.float32),
               jax.ShapeDtypeStruct((_N, _H), jnp.float32)),
)


def _gru_body(relu, part_ref, h_ref, wr_ref, wz_ref, wn_ref, ur_ref, uz_ref,
              un_ref, br_ref, bz_ref, bn_ref, cr_ref, cz_ref, cn_ref, wg_ref,
              hout_ref, mout_ref):
    agg = part_ref[0, : _N, :] + part_ref[1, : _N, :]
    h = h_ref[...]
    r = jax.nn.sigmoid(_mmT(agg, wr_ref[...]) + br_ref[...]
                       + _mmT(h, ur_ref[...]) + cr_ref[...])
    z = jax.nn.sigmoid(_mmT(agg, wz_ref[...]) + bz_ref[...]
                       + _mmT(h, uz_ref[...]) + cz_ref[...])
    n = jnp.tanh(_mmT(agg, wn_ref[...]) + bn_ref[...]
                 + r * (_mmT(h, un_ref[...]) + cn_ref[...]))
    hn = (1.0 - z) * n + z * h
    if relu:
        hn = jnp.maximum(hn, 0.0)
    hout_ref[...] = hn
    mout_ref[...] = jnp.dot(hn, wg_ref[...], preferred_element_type=jnp.float32)


_gru = {
    relu: pl.pallas_call(
        functools.partial(_gru_body, relu),
        out_shape=(jax.ShapeDtypeStruct((_N, _H), jnp.float32),
                   jax.ShapeDtypeStruct((_N, _H), jnp.float32)),
    )
    for relu in (False, True)
}


def _pool_body(h_ref, batch_ref, w1_ref, b1_ref, w2_ref, b2_ref, w3_ref,
               b3_ref, out_ref):
    h = h_ref[...]
    gids = lax.broadcasted_iota(jnp.int32, (_N, _G), 1)
    oh = (batch_ref[...] == gids).astype(jnp.float32)
    sums_t = lax.dot_general(h, oh, (((0,), (0,)), ((), ())),
                             preferred_element_type=jnp.float32)
    counts = jnp.sum(oh, axis=0, keepdims=True)
    pooled_t = sums_t / jnp.maximum(counts, 1.0)
    o1 = jnp.dot(w1_ref[...], pooled_t, preferred_element_type=jnp.float32) + b1_ref[...]
    o2 = jnp.dot(w2_ref[...], o1, preferred_element_type=jnp.float32) + b2_ref[...]
    o3 = jnp.dot(w3_ref[...], o2, preferred_element_type=jnp.float32) + b3_ref[...]
    out_ref[...] = jax.nn.sigmoid(o3)


_pool = pl.pallas_call(
    _pool_body,
    out_shape=jax.ShapeDtypeStruct((1, _G), jnp.float32),
)


def kernel(x, edge_index, batch, W_in, b_in, Wg, Wih, Whh, bih, bhh,
           W1, b1, W2, b2, W3, b3):
    src, dst = edge_index[0], edge_index[1]
    pad = _E_PAD - _E
    src3 = jnp.concatenate([src, jnp.zeros((pad,), jnp.int32)]).reshape(_NW, _CH, _C)
    dst3 = jnp.concatenate([dst, jnp.full((pad,), _N, jnp.int32)]).reshape(_NW, _CH, _C)
    zeros = jnp.zeros((_STRIPE, _H), jnp.float32)
    batch2 = batch.reshape(_N, 1)

    h, m = _in_proj(x, W_in, b_in.reshape(1, _H), Wg[0, 0])
    for j in range(_NB * _NL):
        b_idx, l_idx = divmod(j, _NL)
        part = _seg_sum(src3, dst3, m, zeros)
        if j < _NB * _NL - 1:
            nb, nl = divmod(j + 1, _NL)
            wg_next = Wg[nb, nl]
        else:
            wg_next = Wg[0, 0]
        wih, whh = Wih[b_idx], Whh[b_idx]
        bi, bh = bih[b_idx], bhh[b_idx]
        h, m = _gru[l_idx == _NL - 1](
            part, h,
            wih[0:_H], wih[_H:2 * _H], wih[2 * _H:],
            whh[0:_H], whh[_H:2 * _H], whh[2 * _H:],
            bi[0:_H].reshape(1, _H), bi[_H:2 * _H].reshape(1, _H),
            bi[2 * _H:].reshape(1, _H),
            bh[0:_H].reshape(1, _H), bh[_H:2 * _H].reshape(1, _H),
            bh[2 * _H:].reshape(1, _H),
            wg_next,
        )

    out_t = _pool(h, batch2, W1, b1.reshape(_H, 1), W2, b2.reshape(32, 1),
                  W3, b3.reshape(1, 1))
    return out_t.reshape(_G, 1)


# speedup vs baseline: 7.4005x; 1.2547x over previous
import functools

import jax
import jax.numpy as jnp
from jax import lax
from jax.experimental import pallas as pl
from jax.experimental.pallas import tpu as pltpu
from jax.experimental.pallas import tpu_sc as plsc

_N = 10000
_E = 320000
_D = 128
_H = 64
_G = 256
_NB, _NL = 4, 2

_NC = 2
_NS = 16
_NW = _NC * _NS
_C = 128
_CH = -(-_E // (_NW * _C))
_E_PAD = _NW * _CH * _C
_ACC_N = 10240
_STRIPE = _ACC_N // _NS


def _seg_body(src_hbm, dst_hbm, m_hbm, zero_hbm, out_hbm, srcv, dstv, rows, acc,
              gsem0, gsem1):
    cid = lax.axis_index("c")
    sid = lax.axis_index("s")
    wid = cid * _NS + sid

    pltpu.sync_copy(src_hbm.at[wid], srcv)
    pltpu.sync_copy(dst_hbm.at[wid], dstv)
    pltpu.sync_copy(zero_hbm, acc.at[pl.ds(sid * _STRIPE, _STRIPE)])
    plsc.subcore_barrier()

    b0, b1 = rows.at[0], rows.at[1]
    pltpu.async_copy(m_hbm.at[srcv.at[0]], b0, gsem0)
    pltpu.async_copy(m_hbm.at[srcv.at[1]], b1, gsem1)

    @pl.loop(0, _CH, step=2)
    def _chunks(c):
        pltpu.make_async_copy(m_hbm.at[srcv.at[c]], b0, gsem0).wait()
        pltpu.sync_copy(b0, acc.at[dstv.at[c]], add=True)

        @pl.when(c + 2 < _CH)
        def _issue0():
            pltpu.async_copy(m_hbm.at[srcv.at[c + 2]], b0, gsem0)

        @pl.when(c + 1 < _CH)
        def _odd():
            pltpu.make_async_copy(m_hbm.at[srcv.at[c + 1]], b1, gsem1).wait()
            pltpu.sync_copy(b1, acc.at[dstv.at[c + 1]], add=True)

            @pl.when(c + 3 < _CH)
            def _issue1():
                pltpu.async_copy(m_hbm.at[srcv.at[c + 3]], b1, gsem1)
    plsc.subcore_barrier()
    pltpu.sync_copy(acc.at[pl.ds(sid * _STRIPE, _STRIPE)],
                    out_hbm.at[cid, pl.ds(sid * _STRIPE, _STRIPE)])


_seg_sum = pl.kernel(
    _seg_body,
    out_type=jax.ShapeDtypeStruct((_NC, _ACC_N, _H), jnp.float32),
    mesh=plsc.VectorSubcoreMesh(core_axis_name="c", subcore_axis_name="s"),
    scratch_types=[
        pltpu.VMEM((_CH, _C), jnp.int32),
        pltpu.VMEM((_CH, _C), jnp.int32),
        pltpu.VMEM((2, _C, _H), jnp.float32),
        pltpu.VMEM_SHARED((_ACC_N, _H), jnp.float32),
        pltpu.SemaphoreType.DMA,
        pltpu.SemaphoreType.DMA,
    ],
    compiler_params=pltpu.CompilerParams(use_tc_tiling_on_sc=False),
)


def _mmT(a, w):
    return lax.dot_general(a, w, (((1,), (1,)), ((), ())),
                           preferred_element_type=jnp.float32)


def _in_proj_body(x_ref, win_ref, bin_ref, wg_ref, h_ref, m_ref):
    h = _mmT(x_ref[...], win_ref[...]) + bin_ref[...]
    h_ref[...] = h
    m_ref[...] = jnp.dot(h, wg_ref[...], preferred_element_type=jnp.float32)


_in_proj = pl.pallas_call(
    _in_proj_body,
    out_shape=(jax.ShapeDtypeStruct((_N, _H), jnp.float32),
               jax.ShapeDtypeStruct((_N, _H), jnp.float32)),
)


def _gru_body(relu, part_ref, h_ref, wr_ref, wz_ref, wn_ref, ur_ref, uz_ref,
              un_ref, br_ref, bz_ref, bn_ref, cr_ref, cz_ref, cn_ref, wg_ref,
              hout_ref, mout_ref):
    agg = part_ref[0, : _N, :] + part_ref[1, : _N, :]
    h = h_ref[...]
    r = jax.nn.sigmoid(_mmT(agg, wr_ref[...]) + br_ref[...]
                       + _mmT(h, ur_ref[...]) + cr_ref[...])
    z = jax.nn.sigmoid(_mmT(agg, wz_ref[...]) + bz_ref[...]
                       + _mmT(h, uz_ref[...]) + cz_ref[...])
    n = jnp.tanh(_mmT(agg, wn_ref[...]) + bn_ref[...]
                 + r * (_mmT(h, un_ref[...]) + cn_ref[...]))
    hn = (1.0 - z) * n + z * h
    if relu:
        hn = jnp.maximum(hn, 0.0)
    hout_ref[...] = hn
    mout_ref[...] = jnp.dot(hn, wg_ref[...], preferred_element_type=jnp.float32)


_gru = {
    relu: pl.pallas_call(
        functools.partial(_gru_body, relu),
        out_shape=(jax.ShapeDtypeStruct((_N, _H), jnp.float32),
                   jax.ShapeDtypeStruct((_N, _H), jnp.float32)),
    )
    for relu in (False, True)
}


def _pool_body(h_ref, batch_ref, w1_ref, b1_ref, w2_ref, b2_ref, w3_ref,
               b3_ref, out_ref):
    h = h_ref[...]
    gids = lax.broadcasted_iota(jnp.int32, (_N, _G), 1)
    oh = (batch_ref[...] == gids).astype(jnp.float32)
    sums_t = lax.dot_general(h, oh, (((0,), (0,)), ((), ())),
                             preferred_element_type=jnp.float32)
    counts = jnp.sum(oh, axis=0, keepdims=True)
    pooled_t = sums_t / jnp.maximum(counts, 1.0)
    o1 = jnp.dot(w1_ref[...], pooled_t, preferred_element_type=jnp.float32) + b1_ref[...]
    o2 = jnp.dot(w2_ref[...], o1, preferred_element_type=jnp.float32) + b2_ref[...]
    o3 = jnp.dot(w3_ref[...], o2, preferred_element_type=jnp.float32) + b3_ref[...]
    out_ref[...] = jax.nn.sigmoid(o3)


_pool = pl.pallas_call(
    _pool_body,
    out_shape=jax.ShapeDtypeStruct((1, _G), jnp.float32),
)


def kernel(x, edge_index, batch, W_in, b_in, Wg, Wih, Whh, bih, bhh,
           W1, b1, W2, b2, W3, b3):
    src, dst = edge_index[0], edge_index[1]
    pad = _E_PAD - _E
    src3 = jnp.concatenate([src, jnp.zeros((pad,), jnp.int32)]).reshape(_NW, _CH, _C)
    dst3 = jnp.concatenate([dst, jnp.full((pad,), _N, jnp.int32)]).reshape(_NW, _CH, _C)
    zeros = jnp.zeros((_STRIPE, _H), jnp.float32)
    batch2 = batch.reshape(_N, 1)

    h, m = _in_proj(x, W_in, b_in.reshape(1, _H), Wg[0, 0])
    for j in range(_NB * _NL):
        b_idx, l_idx = divmod(j, _NL)
        part = _seg_sum(src3, dst3, m, zeros)
        if j < _NB * _NL - 1:
            nb, nl = divmod(j + 1, _NL)
            wg_next = Wg[nb, nl]
        else:
            wg_next = Wg[0, 0]
        wih, whh = Wih[b_idx], Whh[b_idx]
        bi, bh = bih[b_idx], bhh[b_idx]
        h, m = _gru[l_idx == _NL - 1](
            part, h,
            wih[0:_H], wih[_H:2 * _H], wih[2 * _H:],
            whh[0:_H], whh[_H:2 * _H], whh[2 * _H:],
            bi[0:_H].reshape(1, _H), bi[_H:2 * _H].reshape(1, _H),
            bi[2 * _H:].reshape(1, _H),
            bh[0:_H].reshape(1, _H), bh[_H:2 * _H].reshape(1, _H),
            bh[2 * _H:].reshape(1, _H),
            wg_next,
        )

    out_t = _pool(h, batch2, W1, b1.reshape(_H, 1), W2, b2.reshape(32, 1),
                  W3, b3.reshape(1, 1))
    return out_t.reshape(_G, 1)
